# baseline (speedup 1.0000x reference)
import jax
import jax.numpy as jnp
from jax.experimental import pallas as pl
from jax.experimental.pallas import tpu as pltpu

N = 4096
H = 256
F32 = jnp.float32


def _feat_body(f0_ref, f1_ref, W0_ref, b0_ref, W1_ref, b1_ref,
               Wg0_ref, Wg1_ref, x0_ref, x1_ref):
    h0 = jnp.dot(f0_ref[...], W0_ref[...], preferred_element_type=F32) + b0_ref[...]
    h1 = jnp.dot(f1_ref[...], W1_ref[...], preferred_element_type=F32) + b1_ref[...]
    h = jnp.concatenate([h0, h1], axis=0)
    x0_ref[...] = jnp.dot(h, Wg0_ref[...], preferred_element_type=F32)
    x1_ref[...] = jnp.dot(h, Wg1_ref[...], preferred_element_type=F32)


def _strip_body(adj_ref, g0_ref, g1_ref, x0_ref, x1_ref, w_ref,
                u0_ref, u1_ref, r_ref):
    k = pl.program_id(0)
    w0 = w_ref[0, 0]
    w1 = w_ref[0, 1]
    BF = jnp.bfloat16
    g0 = g0_ref[...]
    g1 = g1_ref[...]
    v0 = (w0 / jnp.maximum(jnp.sum(g0, axis=0), 1e-12))[:, None]
    v1 = (w1 / jnp.maximum(jnp.sum(g1, axis=0), 1e-12))[:, None]
    x1 = x1_ref[...]
    g0b = g0.astype(BF)
    g1b = g1.astype(BF)
    y0 = (x1 * v0).astype(BF)
    y1 = (x1 * v1).astype(BF)
    v0b = v0.astype(BF)
    v1b = v1.astype(BF)

    @pl.when(k == 0)
    def _():
        u0_ref[...] = jnp.zeros_like(u0_ref)
        u1_ref[...] = jnp.zeros_like(u1_ref)
        r_ref[...] = jnp.zeros_like(r_ref)

    u0_ref[...] += jnp.dot(adj_ref[...].astype(BF), x0_ref[...].astype(BF),
                           preferred_element_type=F32)
    u1_ref[...] += (jnp.dot(g0b, y0, preferred_element_type=F32)
                    + jnp.dot(g1b, y1, preferred_element_type=F32))
    r_ref[...] += (jnp.dot(g0b, v0b, preferred_element_type=F32)
                   + jnp.dot(g1b, v1b, preferred_element_type=F32))


def _att_body(u0_ref, u1_ref, r_ref, Watt_ref, batt_ref, q_ref, Wout_ref,
              logits_ref, hout_ref):
    z0 = jnp.maximum(u0_ref[...], 0.0)
    r = jnp.maximum(r_ref[...], 1e-12)
    z1 = jnp.maximum(u1_ref[...] / r, 0.0)
    Watt = Watt_ref[...]
    batt = batt_ref[...]
    q = q_ref[...]
    s0 = jnp.tanh(jnp.dot(z0, Watt, preferred_element_type=F32) + batt)
    s1 = jnp.tanh(jnp.dot(z1, Watt, preferred_element_type=F32) + batt)
    e0 = jnp.mean(jnp.dot(s0, q, preferred_element_type=F32))
    e1 = jnp.mean(jnp.dot(s1, q, preferred_element_type=F32))
    m = jnp.maximum(e0, e1)
    a0 = jnp.exp(e0 - m)
    a1 = jnp.exp(e1 - m)
    inv = 1.0 / (a0 + a1)
    hout = (a0 * inv) * z0 + (a1 * inv) * z1
    hout_ref[...] = hout
    logits_ref[...] = jnp.dot(hout, Wout_ref[...], preferred_element_type=F32)


def kernel(feat0, feat1, G0, G1, ADJ, type_mask, W0, b0, W1, b1, ch_w,
           Wg0, Wg1, Watt, batt, q_att, Wout, *, interpret=False):
    del type_mask

    w = jax.nn.softmax(ch_w.reshape(2), axis=0).reshape(1, 2)

    x0, x1 = pl.pallas_call(
        _feat_body,
        out_shape=[
            jax.ShapeDtypeStruct((N, H), F32),
            jax.ShapeDtypeStruct((N, H), F32),
        ],
        interpret=interpret,
    )(feat0, feat1, W0, b0.reshape(1, H), W1, b1.reshape(1, H), Wg0, Wg1)

    BK = 256
    nk = N // BK
    u0, u1, r = pl.pallas_call(
        _strip_body,
        grid=(nk,),
        in_specs=[
            pl.BlockSpec((N, BK), lambda k: (0, k)),
            pl.BlockSpec((N, BK), lambda k: (0, k)),
            pl.BlockSpec((N, BK), lambda k: (0, k)),
            pl.BlockSpec((BK, H), lambda k: (k, 0)),
            pl.BlockSpec((BK, H), lambda k: (k, 0)),
            pl.BlockSpec((1, 2), lambda k: (0, 0)),
        ],
        out_specs=[
            pl.BlockSpec((N, H), lambda k: (0, 0)),
            pl.BlockSpec((N, H), lambda k: (0, 0)),
            pl.BlockSpec((N, 1), lambda k: (0, 0)),
        ],
        out_shape=[
            jax.ShapeDtypeStruct((N, H), F32),
            jax.ShapeDtypeStruct((N, H), F32),
            jax.ShapeDtypeStruct((N, 1), F32),
        ],
        compiler_params=pltpu.CompilerParams(
            dimension_semantics=("arbitrary",)),
        interpret=interpret,
    )(ADJ, G0, G1, x0, x1, w)

    logits, h_out = pl.pallas_call(
        _att_body,
        out_shape=[
            jax.ShapeDtypeStruct((N, Wout.shape[1]), F32),
            jax.ShapeDtypeStruct((N, H), F32),
        ],
        interpret=interpret,
    )(u0, u1, r, Watt, batt.reshape(1, -1), q_att.reshape(-1, 1), Wout)

    return (logits, h_out)


# speedup vs baseline: 1.1385x; 1.1385x over previous
import jax
import jax.numpy as jnp
from jax.experimental import pallas as pl
from jax.experimental.pallas import tpu as pltpu

N = 4096
N0 = 2048
H = 256
F32 = jnp.float32


def _han_body(adj_ref, g0_ref, g1_ref, f0_ref, f1_ref, W0_ref, b0_ref,
              W1_ref, b1_ref, Wg0_ref, Wg1_ref, w_ref, Watt_ref, batt_ref,
              q_ref, Wout_ref, logits_ref, hout_ref,
              u0_ref, u1_ref, r_ref):
    k = pl.program_id(0)
    nk = pl.num_programs(0)
    n0_blocks = nk // 2

    h0 = jnp.dot(f0_ref[...], W0_ref[...], preferred_element_type=F32) + b0_ref[...]
    h1 = jnp.dot(f1_ref[...], W1_ref[...], preferred_element_type=F32) + b1_ref[...]
    h = jax.lax.select(k < n0_blocks, h0, h1)
    x0 = jnp.dot(h, Wg0_ref[...], preferred_element_type=F32)
    x1 = jnp.dot(h, Wg1_ref[...], preferred_element_type=F32)

    w0 = w_ref[0, 0]
    w1 = w_ref[0, 1]
    g0 = g0_ref[...]
    g1 = g1_ref[...]
    v0 = (w0 / jnp.maximum(jnp.sum(g0, axis=0), 1e-12))[:, None]
    v1 = (w1 / jnp.maximum(jnp.sum(g1, axis=0), 1e-12))[:, None]
    y0 = x1 * v0
    y1 = x1 * v1

    @pl.when(k == 0)
    def _():
        u0_ref[...] = jnp.zeros_like(u0_ref)
        u1_ref[...] = jnp.zeros_like(u1_ref)
        r_ref[...] = jnp.zeros_like(r_ref)

    u0_ref[...] += jnp.dot(adj_ref[...], x0, preferred_element_type=F32)
    u1_ref[...] += (jnp.dot(g0, y0, preferred_element_type=F32)
                    + jnp.dot(g1, y1, preferred_element_type=F32))
    r_ref[...] += (jnp.dot(g0, v0, preferred_element_type=F32)
                   + jnp.dot(g1, v1, preferred_element_type=F32))

    @pl.when(k == nk - 1)
    def _():
        z0 = jnp.maximum(u0_ref[...], 0.0)
        r = jnp.maximum(r_ref[...], 1e-12)
        z1 = jnp.maximum(u1_ref[...] / r, 0.0)
        Watt = Watt_ref[...]
        batt = batt_ref[...]
        q = q_ref[...]
        s0 = jnp.tanh(jnp.dot(z0, Watt, preferred_element_type=F32) + batt)
        s1 = jnp.tanh(jnp.dot(z1, Watt, preferred_element_type=F32) + batt)
        e0 = jnp.mean(jnp.dot(s0, q, preferred_element_type=F32))
        e1 = jnp.mean(jnp.dot(s1, q, preferred_element_type=F32))
        m = jnp.maximum(e0, e1)
        a0 = jnp.exp(e0 - m)
        a1 = jnp.exp(e1 - m)
        inv = 1.0 / (a0 + a1)
        hout = (a0 * inv) * z0 + (a1 * inv) * z1
        hout_ref[...] = hout
        logits_ref[...] = jnp.dot(hout, Wout_ref[...], preferred_element_type=F32)


def kernel(feat0, feat1, G0, G1, ADJ, type_mask, W0, b0, W1, b1, ch_w,
           Wg0, Wg1, Watt, batt, q_att, Wout, *, interpret=False):
    del type_mask

    w = jax.nn.softmax(ch_w.reshape(2), axis=0).reshape(1, 2)

    BK = 256
    nk = N // BK
    n0b = N0 // BK
    D0 = feat0.shape[1]
    D1 = feat1.shape[1]
    OUT = Wout.shape[1]

    logits, h_out = pl.pallas_call(
        _han_body,
        grid=(nk,),
        in_specs=[
            pl.BlockSpec((N, BK), lambda k: (0, k)),
            pl.BlockSpec((N, BK), lambda k: (0, k)),
            pl.BlockSpec((N, BK), lambda k: (0, k)),
            pl.BlockSpec((BK, D0), lambda k: (jnp.minimum(k, n0b - 1), 0)),
            pl.BlockSpec((BK, D1), lambda k: (jnp.maximum(k - n0b, 0), 0)),
            pl.BlockSpec((D0, H), lambda k: (0, 0)),
            pl.BlockSpec((1, H), lambda k: (0, 0)),
            pl.BlockSpec((D1, H), lambda k: (0, 0)),
            pl.BlockSpec((1, H), lambda k: (0, 0)),
            pl.BlockSpec((H, H), lambda k: (0, 0)),
            pl.BlockSpec((H, H), lambda k: (0, 0)),
            pl.BlockSpec((1, 2), lambda k: (0, 0)),
            pl.BlockSpec((H, Watt.shape[1]), lambda k: (0, 0)),
            pl.BlockSpec((1, Watt.shape[1]), lambda k: (0, 0)),
            pl.BlockSpec((Watt.shape[1], 1), lambda k: (0, 0)),
            pl.BlockSpec((H, OUT), lambda k: (0, 0)),
        ],
        out_specs=[
            pl.BlockSpec((N, OUT), lambda k: (0, 0)),
            pl.BlockSpec((N, H), lambda k: (0, 0)),
        ],
        out_shape=[
            jax.ShapeDtypeStruct((N, OUT), F32),
            jax.ShapeDtypeStruct((N, H), F32),
        ],
        scratch_shapes=[
            pltpu.VMEM((N, H), F32),
            pltpu.VMEM((N, H), F32),
            pltpu.VMEM((N, 1), F32),
        ],
        compiler_params=pltpu.CompilerParams(
            dimension_semantics=("arbitrary",),
            vmem_limit_bytes=100 * 1024 * 1024),
        interpret=interpret,
    )(ADJ, G0, G1, feat0, feat1, W0, b0.reshape(1, H), W1, b1.reshape(1, H),
      Wg0, Wg1, w, Watt, batt.reshape(1, -1), q_att.reshape(-1, 1), Wout)

    return (logits, h_out)


# speedup vs baseline: 1.2572x; 1.1042x over previous
import jax
import jax.numpy as jnp
from jax.experimental import pallas as pl
from jax.experimental.pallas import tpu as pltpu

N = 4096
N0 = 2048
H = 256
F32 = jnp.float32


def _han_body(adj_ref, g0_ref, g1_ref, f0_ref, f1_ref, W0_ref, b0_ref,
              W1_ref, b1_ref, Wg0_ref, Wg1_ref, w_ref, Watt_ref, batt_ref,
              q_ref, Wout_ref, logits_ref, hout_ref,
              u0_ref, u1_ref, r_ref):
    k = pl.program_id(0)
    nk = pl.num_programs(0)
    n0_blocks = nk // 2

    h0 = jnp.dot(f0_ref[...], W0_ref[...], preferred_element_type=F32) + b0_ref[...]
    h1 = jnp.dot(f1_ref[...], W1_ref[...], preferred_element_type=F32) + b1_ref[...]
    h = jax.lax.select(k < n0_blocks, h0, h1)
    x0 = jnp.dot(h, Wg0_ref[...], preferred_element_type=F32)
    x1 = jnp.dot(h, Wg1_ref[...], preferred_element_type=F32)

    cw0 = w_ref[0, 0]
    cw1 = w_ref[0, 1]
    cm = jnp.maximum(cw0, cw1)
    ca0 = jnp.exp(cw0 - cm)
    ca1 = jnp.exp(cw1 - cm)
    cinv = 1.0 / (ca0 + ca1)
    w0 = ca0 * cinv
    w1 = ca1 * cinv
    g0 = g0_ref[...]
    g1 = g1_ref[...]
    v0 = (w0 / jnp.maximum(jnp.sum(g0, axis=0), 1e-12))[:, None]
    v1 = (w1 / jnp.maximum(jnp.sum(g1, axis=0), 1e-12))[:, None]
    y0 = x1 * v0
    y1 = x1 * v1

    @pl.when(k == 0)
    def _():
        u0_ref[...] = jnp.zeros_like(u0_ref)
        u1_ref[...] = jnp.zeros_like(u1_ref)
        r_ref[...] = jnp.zeros_like(r_ref)

    u0_ref[...] += jnp.dot(adj_ref[...], x0, preferred_element_type=F32)
    u1_ref[...] += (jnp.dot(g0, y0, preferred_element_type=F32)
                    + jnp.dot(g1, y1, preferred_element_type=F32))
    r_ref[...] += (jnp.dot(g0, v0, preferred_element_type=F32)
                   + jnp.dot(g1, v1, preferred_element_type=F32))

    @pl.when(k == nk - 1)
    def _():
        z0 = jnp.maximum(u0_ref[...], 0.0)
        r = jnp.maximum(r_ref[...], 1e-12)
        z1 = jnp.maximum(u1_ref[...] / r, 0.0)
        Watt = Watt_ref[...]
        batt = batt_ref[...]
        q = q_ref[...]
        s0 = jnp.tanh(jnp.dot(z0, Watt, preferred_element_type=F32) + batt)
        s1 = jnp.tanh(jnp.dot(z1, Watt, preferred_element_type=F32) + batt)
        n_rows = s0.shape[0]
        e0 = jnp.sum(s0 * q) / n_rows
        e1 = jnp.sum(s1 * q) / n_rows
        m = jnp.maximum(e0, e1)
        a0 = jnp.exp(e0 - m)
        a1 = jnp.exp(e1 - m)
        inv = 1.0 / (a0 + a1)
        hout = (a0 * inv) * z0 + (a1 * inv) * z1
        hout_ref[...] = hout
        logits_ref[...] = jnp.dot(hout, Wout_ref[...], preferred_element_type=F32)


def kernel(feat0, feat1, G0, G1, ADJ, type_mask, W0, b0, W1, b1, ch_w,
           Wg0, Wg1, Watt, batt, q_att, Wout, *, interpret=False):
    del type_mask

    BK = 256
    nk = N // BK
    n0b = N0 // BK
    D0 = feat0.shape[1]
    D1 = feat1.shape[1]
    OUT = Wout.shape[1]

    logits, h_out = pl.pallas_call(
        _han_body,
        grid=(nk,),
        in_specs=[
            pl.BlockSpec((N, BK), lambda k: (0, k)),
            pl.BlockSpec((N, BK), lambda k: (0, k)),
            pl.BlockSpec((N, BK), lambda k: (0, k)),
            pl.BlockSpec((BK, D0), lambda k: (jnp.minimum(k, n0b - 1), 0)),
            pl.BlockSpec((BK, D1), lambda k: (jnp.maximum(k - n0b, 0), 0)),
            pl.BlockSpec((D0, H), lambda k: (0, 0)),
            pl.BlockSpec((1, H), lambda k: (0, 0)),
            pl.BlockSpec((D1, H), lambda k: (0, 0)),
            pl.BlockSpec((1, H), lambda k: (0, 0)),
            pl.BlockSpec((H, H), lambda k: (0, 0)),
            pl.BlockSpec((H, H), lambda k: (0, 0)),
            pl.BlockSpec((1, 2), lambda k: (0, 0)),
            pl.BlockSpec((H, Watt.shape[1]), lambda k: (0, 0)),
            pl.BlockSpec((1, Watt.shape[1]), lambda k: (0, 0)),
            pl.BlockSpec((1, Watt.shape[1]), lambda k: (0, 0)),
            pl.BlockSpec((H, OUT), lambda k: (0, 0)),
        ],
        out_specs=[
            pl.BlockSpec((N, OUT), lambda k: (0, 0)),
            pl.BlockSpec((N, H), lambda k: (0, 0)),
        ],
        out_shape=[
            jax.ShapeDtypeStruct((N, OUT), F32),
            jax.ShapeDtypeStruct((N, H), F32),
        ],
        scratch_shapes=[
            pltpu.VMEM((N, H), F32),
            pltpu.VMEM((N, H), F32),
            pltpu.VMEM((N, 1), F32),
        ],
        compiler_params=pltpu.CompilerParams(
            dimension_semantics=("arbitrary",),
            vmem_limit_bytes=100 * 1024 * 1024),
        interpret=interpret,
    )(ADJ, G0, G1, feat0, feat1, W0, b0.reshape(1, H), W1, b1.reshape(1, H),
      Wg0, Wg1, ch_w.reshape(1, 2), Watt, batt.reshape(1, -1),
      q_att.reshape(1, -1), Wout)

    return (logits, h_out)
